# baseline (speedup 1.0000x reference)
import functools

import jax
import jax.numpy as jnp
from jax import lax
from jax.experimental import pallas as pl
from jax.experimental.pallas import tpu as pltpu
from jax.experimental.pallas import tpu_sc as plsc

_N = 10000
_E = 320000
_G = 64
_H = 64
_DX = 128
_DP = 16
_DE = 16
_W0 = 80

_NC = 2
_NS = 16
_NW = _NC * _NS
_NP = 10112
_RPT = _NP // _NS

_mesh = plsc.VectorSubcoreMesh(core_axis_name="c", subcore_axis_name="s")


def _chunking(chunk):
    nchunks = _E // chunk
    q, r = divmod(nchunks, _NW)
    return q, r


def _acc_zero(zero_hbm, acc_sp, rows_v, r0):
    pltpu.sync_copy(zero_hbm.at[pl.ds(r0, _RPT)], rows_v.at[pl.ds(0, _RPT)])
    pltpu.sync_copy(rows_v.at[pl.ds(0, _RPT)], acc_sp.at[pl.ds(r0, _RPT)])


def _acc_out(acc_sp, out_hbm, rows_v, c, r0):
    pltpu.sync_copy(acc_sp.at[pl.ds(r0, _RPT)], rows_v.at[pl.ds(0, _RPT)])
    pltpu.sync_copy(rows_v.at[pl.ds(0, _RPT)], out_hbm.at[c, pl.ds(r0, _RPT)])


_C0 = 640
_C = 1280


def _gs0_body(h80_hbm, ea_hbm, send_hbm, rec_hbm, z80_hbm, z16_hbm,
              gp_hbm, eap_hbm, acc80_sp, acc16_sp, idx_g, idx_s,
              rows80_v, rows16_v):
    c = lax.axis_index("c")
    s = lax.axis_index("s")
    wid = c * _NS + s
    r0 = s * _RPT

    _acc_zero(z80_hbm, acc80_sp, rows80_v, r0)
    _acc_zero(z16_hbm, acc16_sp, rows16_v, r0)
    plsc.subcore_barrier()

    q, r = _chunking(_C0)
    nch = q + jnp.where(wid < r, 1, 0)
    ch0 = wid * q + jnp.minimum(wid, r)

    def body(j, carry):
        base = (ch0 + j) * _C0
        pltpu.sync_copy(rec_hbm.at[pl.ds(base, _C0)], idx_s)
        pltpu.sync_copy(send_hbm.at[pl.ds(base, _C0)], idx_g)
        pltpu.sync_copy(h80_hbm.at[idx_g], rows80_v)
        pltpu.sync_copy(rows80_v, acc80_sp.at[idx_s], add=True)
        pltpu.sync_copy(ea_hbm.at[pl.ds(base, _C0)], rows16_v)
        pltpu.sync_copy(rows16_v, acc16_sp.at[idx_s], add=True)
        return carry

    lax.fori_loop(0, nch, body, 0)
    plsc.subcore_barrier()
    _acc_out(acc80_sp, gp_hbm, rows80_v, c, r0)
    _acc_out(acc16_sp, eap_hbm, rows16_v, c, r0)


_gs0_call = pl.kernel(
    _gs0_body,
    out_type=(jax.ShapeDtypeStruct((_NC, _NP, _W0), jnp.float32),
              jax.ShapeDtypeStruct((_NC, _NP, _DE), jnp.float32)),
    mesh=_mesh,
    scratch_types=[
        pltpu.VMEM_SHARED((_NP, _W0), jnp.float32),
        pltpu.VMEM_SHARED((_NP, _DE), jnp.float32),
        pltpu.VMEM((_C0,), jnp.int32),
        pltpu.VMEM((_C0,), jnp.int32),
        pltpu.VMEM((_C0, _W0), jnp.float32),
        pltpu.VMEM((_C0, _DE), jnp.float32),
    ],
    compiler_params=pltpu.CompilerParams(use_tc_tiling_on_sc=False),
    name="sc_gs0",
)


def _gs_body(h_hbm, send_hbm, rec_hbm, zero_hbm, out_hbm, acc_sp,
             idx_g, idx_s, rows_v):
    c = lax.axis_index("c")
    s = lax.axis_index("s")
    wid = c * _NS + s
    r0 = s * _RPT

    _acc_zero(zero_hbm, acc_sp, rows_v, r0)
    plsc.subcore_barrier()

    q, r = _chunking(_C)
    nch = q + jnp.where(wid < r, 1, 0)
    ch0 = wid * q + jnp.minimum(wid, r)

    def body(j, carry):
        base = (ch0 + j) * _C
        pltpu.sync_copy(rec_hbm.at[pl.ds(base, _C)], idx_s)
        pltpu.sync_copy(send_hbm.at[pl.ds(base, _C)], idx_g)
        pltpu.sync_copy(h_hbm.at[idx_g], rows_v)
        pltpu.sync_copy(rows_v, acc_sp.at[idx_s], add=True)
        return carry

    lax.fori_loop(0, nch, body, 0)
    plsc.subcore_barrier()
    _acc_out(acc_sp, out_hbm, rows_v, c, r0)


_gs_call = pl.kernel(
    _gs_body,
    out_type=jax.ShapeDtypeStruct((_NC, _NP, _H), jnp.float32),
    mesh=_mesh,
    scratch_types=[
        pltpu.VMEM_SHARED((_NP, _H), jnp.float32),
        pltpu.VMEM((_C,), jnp.int32),
        pltpu.VMEM((_C,), jnp.int32),
        pltpu.VMEM((_C, _H), jnp.float32),
    ],
    compiler_params=pltpu.CompilerParams(use_tc_tiling_on_sc=False),
    name="sc_gs",
)


_f32 = jnp.float32
_RB = 2000
_NB = _N // _RB


def _dot(a, b):
    return jnp.dot(a, b, preferred_element_type=jnp.float32,
                   precision=lax.Precision.HIGHEST)


def _embed_body(x_ref, p_ref, w_ref, b_ref, o_ref):
    w = w_ref[...]
    h = _dot(x_ref[...], w[0:_DX]) + _dot(p_ref[...], w[_DX:]) + b_ref[...]
    o_ref[:, 0:_H] = h
    o_ref[:, _H:_W0] = jnp.concatenate(
        [jnp.ones((_RB, 1), _f32), jnp.zeros((_RB, _W0 - _H - 1), _f32)],
        axis=1)


def _update(h, se, deg, g, mw, hw, mb, hb):
    dh = deg * h
    agg = (_dot(g, mw[0:_H]) + _dot(dh, mw[_H:2 * _H])
           + _dot(se, mw[2 * _H:]) + deg * mb)
    hn = _dot(h, hw[0:_H]) + _dot(agg, hw[_H:]) + hb
    return hn, dh


def _layer0_body(h80_ref, gp_ref, eap_ref, ew_ref, eb_ref, mw_ref, hw_ref,
                 uw_ref, mb_ref, hb_ref, ub_ref, hn_ref, sn_ref, deg_ref):
    gfull = gp_ref[0] + gp_ref[1]
    g = gfull[:, 0:_H]
    deg = gfull[:, _H:_H + 1]
    h = h80_ref[:, 0:_H]
    ea = eap_ref[0] + eap_ref[1]
    se = _dot(ea, ew_ref[...]) + deg * eb_ref[...]
    hn, dh = _update(h, se, deg, g,
                     mw_ref[...], hw_ref[...], mb_ref[...], hb_ref[...])
    uw = uw_ref[...]
    sn_ref[...] = (_dot(g, uw[0:_H]) + _dot(dh, uw[_H:2 * _H])
                   + _dot(se, uw[2 * _H:]) + deg * ub_ref[...])
    hn_ref[...] = hn
    deg_ref[...] = deg


def _layer_body(h_ref, se_ref, deg_ref, gp_ref, mw_ref, hw_ref, uw_ref,
                mb_ref, hb_ref, ub_ref, hn_ref, sn_ref):
    g = gp_ref[0] + gp_ref[1]
    deg = deg_ref[...]
    se = se_ref[...]
    hn, dh = _update(h_ref[...], se, deg, g, mw_ref[...], hw_ref[...],
                     mb_ref[...], hb_ref[...])
    uw = uw_ref[...]
    sn_ref[...] = (_dot(g, uw[0:_H]) + _dot(dh, uw[_H:2 * _H])
                   + _dot(se, uw[2 * _H:]) + deg * ub_ref[...])
    hn_ref[...] = hn


def _final_body(h_ref, se_ref, deg_ref, gp_ref, batch_ref, mw_ref, hw_ref,
                mb_ref, hb_ref, o_ref):
    g = gp_ref[0] + gp_ref[1]
    hn, _ = _update(h_ref[...], se_ref[...], deg_ref[...], g, mw_ref[...],
                    hw_ref[...], mb_ref[...], hb_ref[...])
    ids = lax.broadcasted_iota(jnp.int32, (_G, _RB), 0)
    onehot = (ids == batch_ref[0]).astype(jnp.float32)

    @pl.when(pl.program_id(0) == 0)
    def _():
        o_ref[...] = jnp.zeros_like(o_ref)

    o_ref[...] += _dot(onehot, hn)


def _rows(width):
    return pl.BlockSpec((_RB, width), lambda i: (i, 0))


def _full(*shape):
    return pl.BlockSpec(shape, lambda i: (0,) * len(shape))


def _part(width):
    return pl.BlockSpec((_NC, _RB, width), lambda i: (0, i, 0))


def _tc(body, in_specs, out_specs, out_shapes):
    return pl.pallas_call(body, grid=(_NB,), in_specs=in_specs,
                          out_specs=out_specs, out_shape=out_shapes)


def kernel(x, p, edge_attr, edge_index, batch, embed_W, embed_b, edge_W,
           edge_b, msg_W, msg_b, hup_W, hup_b, eup_W, eup_b):
    send = edge_index[0]
    rec = edge_index[1]
    z80 = jnp.zeros((_NP, _W0), _f32)
    z64 = jnp.zeros((_NP, _H), _f32)
    z16 = jnp.zeros((_NP, _DE), _f32)
    batch3d = batch.reshape(_NB, 1, _RB)

    h80 = _tc(_embed_body,
              [_rows(_DX), _rows(_DP), _full(_DX + _DP, _H), _full(1, _H)],
              _rows(_W0), jax.ShapeDtypeStruct((_N, _W0), _f32))(
        x, p, embed_W, embed_b.reshape(1, _H))

    gp0, eap = _gs0_call(h80, edge_attr, send, rec, z80, z16)

    nsd = (jax.ShapeDtypeStruct((_N, _H), _f32),
           jax.ShapeDtypeStruct((_N, _H), _f32),
           jax.ShapeDtypeStruct((_N, 1), _f32))
    h, se, deg = _tc(
        _layer0_body,
        [_rows(_W0), _part(_W0), _part(_DE), _full(_DE, _H), _full(1, _H),
         _full(3 * _H, _H), _full(2 * _H, _H), _full(3 * _H, _H),
         _full(1, _H), _full(1, _H), _full(1, _H)],
        (_rows(_H), _rows(_H), _rows(1)), nsd)(
        h80, gp0, eap, edge_W, edge_b.reshape(1, _H),
        msg_W[0], hup_W[0], eup_W[0], msg_b[0].reshape(1, _H),
        hup_b[0].reshape(1, _H), eup_b[0].reshape(1, _H))

    gp = _gs_call(h, send, rec, z64)
    h, se = _tc(
        _layer_body,
        [_rows(_H), _rows(_H), _rows(1), _part(_H),
         _full(3 * _H, _H), _full(2 * _H, _H), _full(3 * _H, _H),
         _full(1, _H), _full(1, _H), _full(1, _H)],
        (_rows(_H), _rows(_H)),
        (jax.ShapeDtypeStruct((_N, _H), _f32),
         jax.ShapeDtypeStruct((_N, _H), _f32)))(
        h, se, deg, gp, msg_W[1], hup_W[1], eup_W[1],
        msg_b[1].reshape(1, _H), hup_b[1].reshape(1, _H),
        eup_b[1].reshape(1, _H))

    gp = _gs_call(h, send, rec, z64)
    out = _tc(
        _final_body,
        [_rows(_H), _rows(_H), _rows(1), _part(_H),
         pl.BlockSpec((1, 1, _RB), lambda i: (i, 0, 0)),
         _full(3 * _H, _H), _full(2 * _H, _H), _full(1, _H), _full(1, _H)],
        _full(_G, _H), jax.ShapeDtypeStruct((_G, _H), _f32))(
        h, se, deg, gp, batch3d, msg_W[2], hup_W[2],
        msg_b[2].reshape(1, _H), hup_b[2].reshape(1, _H))
    return out


# speedup vs baseline: 1.4257x; 1.4257x over previous
import functools

import jax
import jax.numpy as jnp
from jax import lax
from jax.experimental import pallas as pl
from jax.experimental.pallas import tpu as pltpu
from jax.experimental.pallas import tpu_sc as plsc

_N = 10000
_E = 320000
_G = 64
_H = 64
_DX = 128
_DP = 16
_DE = 16
_W0 = 80

_NC = 2
_NS = 16
_NW = _NC * _NS
_NP = 10112
_RPT = _NP // _NS

_mesh = plsc.VectorSubcoreMesh(core_axis_name="c", subcore_axis_name="s")


def _chunking(chunk):
    nchunks = _E // chunk
    q, r = divmod(nchunks, _NW)
    return q, r


def _acc_zero(zero_hbm, acc_sp, rows_v, r0):
    pltpu.sync_copy(zero_hbm.at[pl.ds(r0, _RPT)], rows_v.at[pl.ds(0, _RPT)])
    pltpu.sync_copy(rows_v.at[pl.ds(0, _RPT)], acc_sp.at[pl.ds(r0, _RPT)])


def _acc_out(acc_sp, out_hbm, rows_v, c, r0):
    pltpu.sync_copy(acc_sp.at[pl.ds(r0, _RPT)], rows_v.at[pl.ds(0, _RPT)])
    pltpu.sync_copy(rows_v.at[pl.ds(0, _RPT)], out_hbm.at[c, pl.ds(r0, _RPT)])


_C0 = 640
_C = 1280


def _gs0_body(h80_hbm, send_hbm, rec_hbm, z80_hbm, gp_hbm, acc80_sp,
              idx_g, idx_s, rows80_v):
    c = lax.axis_index("c")
    s = lax.axis_index("s")
    wid = c * _NS + s
    r0 = s * _RPT

    _acc_zero(z80_hbm, acc80_sp, rows80_v, r0)
    plsc.subcore_barrier()

    q, r = _chunking(_C0)
    nch = q + jnp.where(wid < r, 1, 0)
    ch0 = wid * q + jnp.minimum(wid, r)

    def body(j, carry):
        base = (ch0 + j) * _C0
        pltpu.sync_copy(rec_hbm.at[pl.ds(base, _C0)], idx_s)
        pltpu.sync_copy(send_hbm.at[pl.ds(base, _C0)], idx_g)
        pltpu.sync_copy(h80_hbm.at[idx_g], rows80_v)
        pltpu.sync_copy(rows80_v, acc80_sp.at[idx_s], add=True)
        return carry

    lax.fori_loop(0, nch, body, 0)
    plsc.subcore_barrier()
    _acc_out(acc80_sp, gp_hbm, rows80_v, c, r0)


_gs0_call = pl.kernel(
    _gs0_body,
    out_type=jax.ShapeDtypeStruct((_NC, _NP, _W0), jnp.float32),
    mesh=_mesh,
    scratch_types=[
        pltpu.VMEM_SHARED((_NP, _W0), jnp.float32),
        pltpu.VMEM((_C0,), jnp.int32),
        pltpu.VMEM((_C0,), jnp.int32),
        pltpu.VMEM((_C0, _W0), jnp.float32),
    ],
    compiler_params=pltpu.CompilerParams(use_tc_tiling_on_sc=False),
    name="sc_gs0",
)


def _ea_body(ea_hbm, rec_hbm, z16_hbm, eap_hbm, acc16_sp, idx_s, rows16_v):
    c = lax.axis_index("c")
    s = lax.axis_index("s")
    wid = c * _NS + s
    r0 = s * _RPT

    _acc_zero(z16_hbm, acc16_sp, rows16_v, r0)
    plsc.subcore_barrier()

    q, r = _chunking(_C)
    nch = q + jnp.where(wid < r, 1, 0)
    ch0 = wid * q + jnp.minimum(wid, r)

    def body(j, carry):
        base = (ch0 + j) * _C
        pltpu.sync_copy(rec_hbm.at[pl.ds(base, _C)], idx_s)
        pltpu.sync_copy(ea_hbm.at[pl.ds(base, _C)], rows16_v)
        pltpu.sync_copy(rows16_v, acc16_sp.at[idx_s], add=True)
        return carry

    lax.fori_loop(0, nch, body, 0)
    plsc.subcore_barrier()
    _acc_out(acc16_sp, eap_hbm, rows16_v, c, r0)


_ea_call = pl.kernel(
    _ea_body,
    out_type=jax.ShapeDtypeStruct((_NC, _NP, _DE), jnp.float32),
    mesh=_mesh,
    scratch_types=[
        pltpu.VMEM_SHARED((_NP, _DE), jnp.float32),
        pltpu.VMEM((_C,), jnp.int32),
        pltpu.VMEM((_C, _DE), jnp.float32),
    ],
    compiler_params=pltpu.CompilerParams(use_tc_tiling_on_sc=False),
    name="sc_ea",
)


def _gs_body(h_hbm, send_hbm, rec_hbm, zero_hbm, out_hbm, acc_sp,
             idx_g, idx_s, rows_v):
    c = lax.axis_index("c")
    s = lax.axis_index("s")
    wid = c * _NS + s
    r0 = s * _RPT

    _acc_zero(zero_hbm, acc_sp, rows_v, r0)
    plsc.subcore_barrier()

    q, r = _chunking(_C)
    nch = q + jnp.where(wid < r, 1, 0)
    ch0 = wid * q + jnp.minimum(wid, r)

    def body(j, carry):
        base = (ch0 + j) * _C
        pltpu.sync_copy(rec_hbm.at[pl.ds(base, _C)], idx_s)
        pltpu.sync_copy(send_hbm.at[pl.ds(base, _C)], idx_g)
        pltpu.sync_copy(h_hbm.at[idx_g], rows_v)
        pltpu.sync_copy(rows_v, acc_sp.at[idx_s], add=True)
        return carry

    lax.fori_loop(0, nch, body, 0)
    plsc.subcore_barrier()
    _acc_out(acc_sp, out_hbm, rows_v, c, r0)


_gs_call = pl.kernel(
    _gs_body,
    out_type=jax.ShapeDtypeStruct((_NC, _NP, _H), jnp.float32),
    mesh=_mesh,
    scratch_types=[
        pltpu.VMEM_SHARED((_NP, _H), jnp.float32),
        pltpu.VMEM((_C,), jnp.int32),
        pltpu.VMEM((_C,), jnp.int32),
        pltpu.VMEM((_C, _H), jnp.float32),
    ],
    compiler_params=pltpu.CompilerParams(use_tc_tiling_on_sc=False),
    name="sc_gs",
)


_f32 = jnp.float32
_RB = 2000
_NB = _N // _RB


def _dot(a, b):
    return jnp.dot(a, b, preferred_element_type=jnp.float32)


def _embed_body(x_ref, p_ref, w_ref, b_ref, o_ref):
    w = w_ref[...]
    h = _dot(x_ref[...], w[0:_DX]) + _dot(p_ref[...], w[_DX:]) + b_ref[...]
    o_ref[:, 0:_H] = h
    o_ref[:, _H:_W0] = jnp.concatenate(
        [jnp.ones((_RB, 1), _f32), jnp.zeros((_RB, _W0 - _H - 1), _f32)],
        axis=1)


def _update(h, se, deg, g, mw, hw, mb, hb):
    dh = deg * h
    agg = (_dot(g, mw[0:_H]) + _dot(dh, mw[_H:2 * _H])
           + _dot(se, mw[2 * _H:]) + deg * mb)
    hn = _dot(h, hw[0:_H]) + _dot(agg, hw[_H:]) + hb
    return hn, dh


def _layer0_body(h80_ref, gp_ref, eap_ref, ew_ref, eb_ref, mw_ref, hw_ref,
                 uw_ref, mb_ref, hb_ref, ub_ref, hn_ref, sn_ref, deg_ref):
    gfull = gp_ref[0] + gp_ref[1]
    g = gfull[:, 0:_H]
    deg = gfull[:, _H:_H + 1]
    h = h80_ref[:, 0:_H]
    ea = eap_ref[0] + eap_ref[1]
    se = _dot(ea, ew_ref[...]) + deg * eb_ref[...]
    hn, dh = _update(h, se, deg, g,
                     mw_ref[...], hw_ref[...], mb_ref[...], hb_ref[...])
    uw = uw_ref[...]
    sn_ref[...] = (_dot(g, uw[0:_H]) + _dot(dh, uw[_H:2 * _H])
                   + _dot(se, uw[2 * _H:]) + deg * ub_ref[...])
    hn_ref[...] = hn
    deg_ref[...] = deg


def _layer_body(h_ref, se_ref, deg_ref, gp_ref, mw_ref, hw_ref, uw_ref,
                mb_ref, hb_ref, ub_ref, hn_ref, sn_ref):
    g = gp_ref[0] + gp_ref[1]
    deg = deg_ref[...]
    se = se_ref[...]
    hn, dh = _update(h_ref[...], se, deg, g, mw_ref[...], hw_ref[...],
                     mb_ref[...], hb_ref[...])
    uw = uw_ref[...]
    sn_ref[...] = (_dot(g, uw[0:_H]) + _dot(dh, uw[_H:2 * _H])
                   + _dot(se, uw[2 * _H:]) + deg * ub_ref[...])
    hn_ref[...] = hn


def _final_body(h_ref, se_ref, deg_ref, gp_ref, batch_ref, mw_ref, hw_ref,
                mb_ref, hb_ref, o_ref):
    g = gp_ref[0] + gp_ref[1]
    hn, _ = _update(h_ref[...], se_ref[...], deg_ref[...], g, mw_ref[...],
                    hw_ref[...], mb_ref[...], hb_ref[...])
    ids = lax.broadcasted_iota(jnp.int32, (_G, _RB), 0)
    onehot = (ids == batch_ref[0]).astype(jnp.float32)

    @pl.when(pl.program_id(0) == 0)
    def _():
        o_ref[...] = jnp.zeros_like(o_ref)

    o_ref[...] += _dot(onehot, hn)


def _rows(width):
    return pl.BlockSpec((_RB, width), lambda i: (i, 0))


def _full(*shape):
    return pl.BlockSpec(shape, lambda i: (0,) * len(shape))


def _part(width):
    return pl.BlockSpec((_NC, _RB, width), lambda i: (0, i, 0))


def _tc(body, in_specs, out_specs, out_shapes):
    return pl.pallas_call(body, grid=(_NB,), in_specs=in_specs,
                          out_specs=out_specs, out_shape=out_shapes)


def kernel(x, p, edge_attr, edge_index, batch, embed_W, embed_b, edge_W,
           edge_b, msg_W, msg_b, hup_W, hup_b, eup_W, eup_b):
    send = edge_index[0]
    rec = edge_index[1]
    z80 = jnp.zeros((_NP, _W0), _f32)
    z64 = jnp.zeros((_NP, _H), _f32)
    z16 = jnp.zeros((_NP, _DE), _f32)
    batch3d = batch.reshape(_NB, 1, _RB)

    h80 = _tc(_embed_body,
              [_rows(_DX), _rows(_DP), _full(_DX + _DP, _H), _full(1, _H)],
              _rows(_W0), jax.ShapeDtypeStruct((_N, _W0), _f32))(
        x, p, embed_W, embed_b.reshape(1, _H))

    gp0 = _gs0_call(h80, send, rec, z80)
    eap = _ea_call(edge_attr, rec, z16)

    nsd = (jax.ShapeDtypeStruct((_N, _H), _f32),
           jax.ShapeDtypeStruct((_N, _H), _f32),
           jax.ShapeDtypeStruct((_N, 1), _f32))
    h, se, deg = _tc(
        _layer0_body,
        [_rows(_W0), _part(_W0), _part(_DE), _full(_DE, _H), _full(1, _H),
         _full(3 * _H, _H), _full(2 * _H, _H), _full(3 * _H, _H),
         _full(1, _H), _full(1, _H), _full(1, _H)],
        (_rows(_H), _rows(_H), _rows(1)), nsd)(
        h80, gp0, eap, edge_W, edge_b.reshape(1, _H),
        msg_W[0], hup_W[0], eup_W[0], msg_b[0].reshape(1, _H),
        hup_b[0].reshape(1, _H), eup_b[0].reshape(1, _H))

    gp = _gs_call(h, send, rec, z64)
    h, se = _tc(
        _layer_body,
        [_rows(_H), _rows(_H), _rows(1), _part(_H),
         _full(3 * _H, _H), _full(2 * _H, _H), _full(3 * _H, _H),
         _full(1, _H), _full(1, _H), _full(1, _H)],
        (_rows(_H), _rows(_H)),
        (jax.ShapeDtypeStruct((_N, _H), _f32),
         jax.ShapeDtypeStruct((_N, _H), _f32)))(
        h, se, deg, gp, msg_W[1], hup_W[1], eup_W[1],
        msg_b[1].reshape(1, _H), hup_b[1].reshape(1, _H),
        eup_b[1].reshape(1, _H))

    gp = _gs_call(h, send, rec, z64)
    out = _tc(
        _final_body,
        [_rows(_H), _rows(_H), _rows(1), _part(_H),
         pl.BlockSpec((1, 1, _RB), lambda i: (i, 0, 0)),
         _full(3 * _H, _H), _full(2 * _H, _H), _full(1, _H), _full(1, _H)],
        _full(_G, _H), jax.ShapeDtypeStruct((_G, _H), _f32))(
        h, se, deg, gp, batch3d, msg_W[2], hup_W[2],
        msg_b[2].reshape(1, _H), hup_b[2].reshape(1, _H))
    return out


# speedup vs baseline: 1.5394x; 1.0797x over previous
import functools

import jax
import jax.numpy as jnp
from jax import lax
from jax.experimental import pallas as pl
from jax.experimental.pallas import tpu as pltpu
from jax.experimental.pallas import tpu_sc as plsc

_N = 10000
_E = 320000
_G = 64
_H = 64
_DX = 128
_DP = 16
_DE = 16
_W0 = 80

_NC = 2
_NS = 16
_NW = _NC * _NS
_NP = 10112
_RPT = _NP // _NS

_mesh = plsc.VectorSubcoreMesh(core_axis_name="c", subcore_axis_name="s")


def _chunking(chunk):
    nchunks = _E // chunk
    q, r = divmod(nchunks, _NW)
    return q, r


def _acc_zero(zero_hbm, acc_sp, rows_v, r0):
    pltpu.sync_copy(zero_hbm.at[pl.ds(r0, _RPT)], rows_v.at[pl.ds(0, _RPT)])
    pltpu.sync_copy(rows_v.at[pl.ds(0, _RPT)], acc_sp.at[pl.ds(r0, _RPT)])


def _acc_out(acc_sp, out_hbm, rows_v, c, r0):
    pltpu.sync_copy(acc_sp.at[pl.ds(r0, _RPT)], rows_v.at[pl.ds(0, _RPT)])
    pltpu.sync_copy(rows_v.at[pl.ds(0, _RPT)], out_hbm.at[c, pl.ds(r0, _RPT)])


_CP = 400
_NCH = _E // _CP // _NW
_C = 1280


def _pipe_loop(table_hbm, send_hbm, rec_hbm, acc_sp, ch0,
               ig0, is0, rb0, sr0, sw0, ig1, is1, rb1, sr1, sw1):
    def idx_load(j, ig, is_):
        base = (ch0 + j) * _CP
        pltpu.sync_copy(send_hbm.at[pl.ds(base, _CP)], ig)
        pltpu.sync_copy(rec_hbm.at[pl.ds(base, _CP)], is_)

    def gather_start(ig, rows, sem):
        pltpu.async_copy(table_hbm.at[ig], rows, sem)

    def gather_wait(ig, rows, sem):
        pltpu.make_async_copy(table_hbm.at[ig], rows, sem).wait()

    def scat_start(is_, rows, sem):
        pltpu.async_copy(rows, acc_sp.at[is_], sem, add=True)

    def scat_wait(is_, rows, sem):
        pltpu.make_async_copy(rows, acc_sp.at[is_], sem).wait()

    idx_load(0, ig0, is0)
    gather_start(ig0, rb0, sr0)

    def body(j, carry):
        m = lax.rem(j, 2)

        @pl.when(m == 1)
        def _():
            @pl.when(j >= 3)
            def _():
                scat_wait(is1, rb1, sw1)
            idx_load(j, ig1, is1)
            gather_start(ig1, rb1, sr1)
            gather_wait(ig0, rb0, sr0)
            scat_start(is0, rb0, sw0)

        @pl.when(m == 0)
        def _():
            scat_wait(is0, rb0, sw0)
            idx_load(j, ig0, is0)
            gather_start(ig0, rb0, sr0)
            gather_wait(ig1, rb1, sr1)
            scat_start(is1, rb1, sw1)

        return carry

    lax.fori_loop(1, _NCH, body, 0)
    gather_wait(ig0, rb0, sr0)
    scat_start(is0, rb0, sw0)
    scat_wait(is1, rb1, sw1)
    scat_wait(is0, rb0, sw0)


_PIECES = tuple((off, min(_CP, _RPT - off)) for off in range(0, _RPT, _CP))


def _acc_zero_p(zero_hbm, acc_sp, rows_v, r0):
    for off, sz in _PIECES:
        pltpu.sync_copy(zero_hbm.at[pl.ds(r0 + off, sz)],
                        rows_v.at[pl.ds(0, sz)])
        pltpu.sync_copy(rows_v.at[pl.ds(0, sz)],
                        acc_sp.at[pl.ds(r0 + off, sz)])


def _acc_out_p(acc_sp, out_hbm, rows_v, c, r0):
    for off, sz in _PIECES:
        pltpu.sync_copy(acc_sp.at[pl.ds(r0 + off, sz)],
                        rows_v.at[pl.ds(0, sz)])
        pltpu.sync_copy(rows_v.at[pl.ds(0, sz)],
                        out_hbm.at[c, pl.ds(r0 + off, sz)])


def _make_gs_kernel(width, name):
    def body(tbl_hbm, send_hbm, rec_hbm, z_hbm, out_hbm, acc_sp,
             ig0, is0, rb0, ig1, is1, rb1, sr0, sw0, sr1, sw1):
        c = lax.axis_index("c")
        s = lax.axis_index("s")
        wid = c * _NS + s
        r0 = s * _RPT

        _acc_zero_p(z_hbm, acc_sp, rb0, r0)
        plsc.subcore_barrier()
        _pipe_loop(tbl_hbm, send_hbm, rec_hbm, acc_sp, wid * _NCH,
                   ig0, is0, rb0, sr0, sw0, ig1, is1, rb1, sr1, sw1)
        plsc.subcore_barrier()
        _acc_out_p(acc_sp, out_hbm, rb0, c, r0)

    return pl.kernel(
        body,
        out_type=jax.ShapeDtypeStruct((_NC, _NP, width), jnp.float32),
        mesh=_mesh,
        scratch_types=[
            pltpu.VMEM_SHARED((_NP, width), jnp.float32),
            pltpu.VMEM((_CP,), jnp.int32),
            pltpu.VMEM((_CP,), jnp.int32),
            pltpu.VMEM((_CP, width), jnp.float32),
            pltpu.VMEM((_CP,), jnp.int32),
            pltpu.VMEM((_CP,), jnp.int32),
            pltpu.VMEM((_CP, width), jnp.float32),
            pltpu.SemaphoreType.DMA,
            pltpu.SemaphoreType.DMA,
            pltpu.SemaphoreType.DMA,
            pltpu.SemaphoreType.DMA,
        ],
        compiler_params=pltpu.CompilerParams(use_tc_tiling_on_sc=False),
        name=name,
    )


_gs0_call = _make_gs_kernel(_W0, "sc_gs0")


def _ea_body(ea_hbm, rec_hbm, z16_hbm, eap_hbm, acc16_sp, idx_s, rows16_v):
    c = lax.axis_index("c")
    s = lax.axis_index("s")
    wid = c * _NS + s
    r0 = s * _RPT

    _acc_zero(z16_hbm, acc16_sp, rows16_v, r0)
    plsc.subcore_barrier()

    q, r = _chunking(_C)
    nch = q + jnp.where(wid < r, 1, 0)
    ch0 = wid * q + jnp.minimum(wid, r)

    def body(j, carry):
        base = (ch0 + j) * _C
        pltpu.sync_copy(rec_hbm.at[pl.ds(base, _C)], idx_s)
        pltpu.sync_copy(ea_hbm.at[pl.ds(base, _C)], rows16_v)
        pltpu.sync_copy(rows16_v, acc16_sp.at[idx_s], add=True)
        return carry

    lax.fori_loop(0, nch, body, 0)
    plsc.subcore_barrier()
    _acc_out(acc16_sp, eap_hbm, rows16_v, c, r0)


_ea_call = pl.kernel(
    _ea_body,
    out_type=jax.ShapeDtypeStruct((_NC, _NP, _DE), jnp.float32),
    mesh=_mesh,
    scratch_types=[
        pltpu.VMEM_SHARED((_NP, _DE), jnp.float32),
        pltpu.VMEM((_C,), jnp.int32),
        pltpu.VMEM((_C, _DE), jnp.float32),
    ],
    compiler_params=pltpu.CompilerParams(use_tc_tiling_on_sc=False),
    name="sc_ea",
)


_gs_call = _make_gs_kernel(_H, "sc_gs")


_f32 = jnp.float32
_RB = 2000
_NB = _N // _RB


def _dot(a, b):
    return jnp.dot(a, b, preferred_element_type=jnp.float32)


def _embed_body(x_ref, p_ref, w_ref, b_ref, o_ref):
    w = w_ref[...]
    h = _dot(x_ref[...], w[0:_DX]) + _dot(p_ref[...], w[_DX:]) + b_ref[...]
    o_ref[:, 0:_H] = h
    o_ref[:, _H:_W0] = jnp.concatenate(
        [jnp.ones((_RB, 1), _f32), jnp.zeros((_RB, _W0 - _H - 1), _f32)],
        axis=1)


def _update(h, se, deg, g, mw, hw, mb, hb):
    dh = deg * h
    agg = (_dot(g, mw[0:_H]) + _dot(dh, mw[_H:2 * _H])
           + _dot(se, mw[2 * _H:]) + deg * mb)
    hn = _dot(h, hw[0:_H]) + _dot(agg, hw[_H:]) + hb
    return hn, dh


def _layer0_body(h80_ref, gp_ref, eap_ref, ew_ref, eb_ref, mw_ref, hw_ref,
                 uw_ref, mb_ref, hb_ref, ub_ref, hn_ref, sn_ref, deg_ref):
    gfull = gp_ref[0] + gp_ref[1]
    g = gfull[:, 0:_H]
    deg = gfull[:, _H:_H + 1]
    h = h80_ref[:, 0:_H]
    ea = eap_ref[0] + eap_ref[1]
    se = _dot(ea, ew_ref[...]) + deg * eb_ref[...]
    hn, dh = _update(h, se, deg, g,
                     mw_ref[...], hw_ref[...], mb_ref[...], hb_ref[...])
    uw = uw_ref[...]
    sn_ref[...] = (_dot(g, uw[0:_H]) + _dot(dh, uw[_H:2 * _H])
                   + _dot(se, uw[2 * _H:]) + deg * ub_ref[...])
    hn_ref[...] = hn
    deg_ref[...] = deg


def _layer_body(h_ref, se_ref, deg_ref, gp_ref, mw_ref, hw_ref, uw_ref,
                mb_ref, hb_ref, ub_ref, hn_ref, sn_ref):
    g = gp_ref[0] + gp_ref[1]
    deg = deg_ref[...]
    se = se_ref[...]
    hn, dh = _update(h_ref[...], se, deg, g, mw_ref[...], hw_ref[...],
                     mb_ref[...], hb_ref[...])
    uw = uw_ref[...]
    sn_ref[...] = (_dot(g, uw[0:_H]) + _dot(dh, uw[_H:2 * _H])
                   + _dot(se, uw[2 * _H:]) + deg * ub_ref[...])
    hn_ref[...] = hn


def _final_body(h_ref, se_ref, deg_ref, gp_ref, batch_ref, mw_ref, hw_ref,
                mb_ref, hb_ref, o_ref):
    g = gp_ref[0] + gp_ref[1]
    hn, _ = _update(h_ref[...], se_ref[...], deg_ref[...], g, mw_ref[...],
                    hw_ref[...], mb_ref[...], hb_ref[...])
    ids = lax.broadcasted_iota(jnp.int32, (_G, _RB), 0)
    onehot = (ids == batch_ref[0]).astype(jnp.float32)

    @pl.when(pl.program_id(0) == 0)
    def _():
        o_ref[...] = jnp.zeros_like(o_ref)

    o_ref[...] += _dot(onehot, hn)


def _rows(width):
    return pl.BlockSpec((_RB, width), lambda i: (i, 0))


def _full(*shape):
    return pl.BlockSpec(shape, lambda i: (0,) * len(shape))


def _part(width):
    return pl.BlockSpec((_NC, _RB, width), lambda i: (0, i, 0))


def _tc(body, in_specs, out_specs, out_shapes):
    return pl.pallas_call(body, grid=(_NB,), in_specs=in_specs,
                          out_specs=out_specs, out_shape=out_shapes)


def kernel(x, p, edge_attr, edge_index, batch, embed_W, embed_b, edge_W,
           edge_b, msg_W, msg_b, hup_W, hup_b, eup_W, eup_b):
    send = edge_index[0]
    rec = edge_index[1]
    z80 = jnp.zeros((_NP, _W0), _f32)
    z64 = jnp.zeros((_NP, _H), _f32)
    z16 = jnp.zeros((_NP, _DE), _f32)
    batch3d = batch.reshape(_NB, 1, _RB)

    h80 = _tc(_embed_body,
              [_rows(_DX), _rows(_DP), _full(_DX + _DP, _H), _full(1, _H)],
              _rows(_W0), jax.ShapeDtypeStruct((_N, _W0), _f32))(
        x, p, embed_W, embed_b.reshape(1, _H))

    gp0 = _gs0_call(h80, send, rec, z80)
    eap = _ea_call(edge_attr, rec, z16)

    nsd = (jax.ShapeDtypeStruct((_N, _H), _f32),
           jax.ShapeDtypeStruct((_N, _H), _f32),
           jax.ShapeDtypeStruct((_N, 1), _f32))
    h, se, deg = _tc(
        _layer0_body,
        [_rows(_W0), _part(_W0), _part(_DE), _full(_DE, _H), _full(1, _H),
         _full(3 * _H, _H), _full(2 * _H, _H), _full(3 * _H, _H),
         _full(1, _H), _full(1, _H), _full(1, _H)],
        (_rows(_H), _rows(_H), _rows(1)), nsd)(
        h80, gp0, eap, edge_W, edge_b.reshape(1, _H),
        msg_W[0], hup_W[0], eup_W[0], msg_b[0].reshape(1, _H),
        hup_b[0].reshape(1, _H), eup_b[0].reshape(1, _H))

    gp = _gs_call(h, send, rec, z64)
    h, se = _tc(
        _layer_body,
        [_rows(_H), _rows(_H), _rows(1), _part(_H),
         _full(3 * _H, _H), _full(2 * _H, _H), _full(3 * _H, _H),
         _full(1, _H), _full(1, _H), _full(1, _H)],
        (_rows(_H), _rows(_H)),
        (jax.ShapeDtypeStruct((_N, _H), _f32),
         jax.ShapeDtypeStruct((_N, _H), _f32)))(
        h, se, deg, gp, msg_W[1], hup_W[1], eup_W[1],
        msg_b[1].reshape(1, _H), hup_b[1].reshape(1, _H),
        eup_b[1].reshape(1, _H))

    gp = _gs_call(h, send, rec, z64)
    out = _tc(
        _final_body,
        [_rows(_H), _rows(_H), _rows(1), _part(_H),
         pl.BlockSpec((1, 1, _RB), lambda i: (i, 0, 0)),
         _full(3 * _H, _H), _full(2 * _H, _H), _full(1, _H), _full(1, _H)],
        _full(_G, _H), jax.ShapeDtypeStruct((_G, _H), _f32))(
        h, se, deg, gp, batch3d, msg_W[2], hup_W[2],
        msg_b[2].reshape(1, _H), hup_b[2].reshape(1, _H))
    return out
